# Initial kernel scaffold; baseline (speedup 1.0000x reference)
import jax
import jax.numpy as jnp
from jax.experimental import pallas as pl


def kernel(x, edge_index, edge_attr, W1, b1, W2, b2, W3, b3):
    raise NotImplementedError("write your pallas kernel here")


# speedup vs baseline: 6.5672x; 6.5672x over previous
import functools

import jax
import jax.numpy as jnp
from jax import lax
from jax.experimental import pallas as pl
from jax.experimental.pallas import tpu as pltpu
from jax.experimental.pallas import tpu_sc as plsc

N = 100000
E = 1600000
NP = 100352
STRIPE = NP // 16
EP = 1601536
B = 128
NBLK = EP // (16 * B)
NBLK2 = EP // (32 * B)
CW = 16
C = 4
D = 64
NT = 102400
BLK = 1024
GRID = NT // BLK

_mesh = plsc.VectorSubcoreMesh(core_axis_name="c", subcore_axis_name="s")


@functools.partial(
    pl.kernel,
    out_type=jax.ShapeDtypeStruct((2, NP), jnp.float32),
    mesh=_mesh,
    compiler_params=pltpu.CompilerParams(use_tc_tiling_on_sc=False),
    scratch_types=[
        pltpu.VMEM_SHARED((NP,), jnp.float32),
        pltpu.VMEM((B,), jnp.int32),
        pltpu.VMEM((B,), jnp.float32),
    ],
)
def _sc_deg(dst_hbm, zeros1_hbm, out_hbm, tbl, dbuf, ones):
    cc = lax.axis_index("c")
    ss = lax.axis_index("s")
    rbase = ss * STRIPE
    for i in range(B // 16):
        ones[pl.ds(i * 16, 16)] = jnp.full((16,), 1.0, jnp.float32)
    pltpu.sync_copy(zeros1_hbm.at[pl.ds(rbase, STRIPE)],
                    tbl.at[pl.ds(rbase, STRIPE)])
    plsc.subcore_barrier()
    ebase = (cc * 16 + ss) * (NBLK2 * B)

    def body(b, carry):
        off = ebase + b * B
        pltpu.sync_copy(dst_hbm.at[pl.ds(off, B)], dbuf)
        pltpu.sync_copy(ones, tbl.at[dbuf], add=True)
        return carry

    lax.fori_loop(0, NBLK2, body, 0)
    plsc.subcore_barrier()
    pltpu.sync_copy(tbl.at[pl.ds(rbase, STRIPE)],
                    out_hbm.at[cc].at[pl.ds(rbase, STRIPE)])


@functools.partial(
    pl.kernel,
    out_type=jax.ShapeDtypeStruct((2, NP), jnp.float32),
    mesh=_mesh,
    compiler_params=pltpu.CompilerParams(use_tc_tiling_on_sc=False),
    scratch_types=[
        pltpu.VMEM_SHARED((NP,), jnp.float32),
        pltpu.VMEM((B,), jnp.int32),
        pltpu.VMEM((B,), jnp.int32),
        pltpu.VMEM((B,), jnp.float32),
        pltpu.SemaphoreType.DMA,
    ],
)
def _sc_s(src_hbm, dst_hbm, dinv_hbm, zeros1_hbm, out_hbm,
          tbl, sbuf, dbuf, dvals, sem):
    cc = lax.axis_index("c")
    ss = lax.axis_index("s")
    rbase = ss * STRIPE
    pltpu.sync_copy(zeros1_hbm.at[pl.ds(rbase, STRIPE)],
                    tbl.at[pl.ds(rbase, STRIPE)])
    plsc.subcore_barrier()
    ebase = (cc * 16 + ss) * (NBLK2 * B)

    def body(b, carry):
        off = ebase + b * B
        pltpu.sync_copy(src_hbm.at[pl.ds(off, B)], sbuf)
        pltpu.sync_copy(dst_hbm.at[pl.ds(off, B)], dbuf)
        pltpu.async_copy(dinv_hbm.at[dbuf], dvals, sem).wait()
        pltpu.sync_copy(dvals, tbl.at[sbuf], add=True)
        return carry

    lax.fori_loop(0, NBLK2, body, 0)
    plsc.subcore_barrier()
    pltpu.sync_copy(tbl.at[pl.ds(rbase, STRIPE)],
                    out_hbm.at[cc].at[pl.ds(rbase, STRIPE)])


@functools.partial(
    pl.kernel,
    out_type=jax.ShapeDtypeStruct((C, NP, CW), jnp.float32),
    mesh=_mesh,
    compiler_params=pltpu.CompilerParams(use_tc_tiling_on_sc=False),
    scratch_types=[
        pltpu.VMEM_SHARED((NP, CW), jnp.float32),
        pltpu.VMEM((B,), jnp.int32),
        pltpu.VMEM((B,), jnp.int32),
        pltpu.VMEM((B, CW), jnp.float32),
        pltpu.SemaphoreType.DMA,
    ],
)
def _sc_agg(g_hbm, src_hbm, dst_hbm, zeros2_hbm, out_hbm,
            tbl, sbuf, dbuf, rows, sem):
    cc = lax.axis_index("c")
    ss = lax.axis_index("s")
    rbase = ss * STRIPE
    for p in range(2):
        k = cc * 2 + p
        pltpu.sync_copy(zeros2_hbm.at[pl.ds(rbase, STRIPE), :],
                        tbl.at[pl.ds(rbase, STRIPE), :])
        plsc.subcore_barrier()
        ebase = ss * (NBLK * B)

        def body(b, carry):
            off = ebase + b * B
            pltpu.sync_copy(src_hbm.at[pl.ds(off, B)], sbuf)
            pltpu.sync_copy(dst_hbm.at[pl.ds(off, B)], dbuf)
            pltpu.async_copy(g_hbm.at[k].at[sbuf], rows, sem).wait()
            pltpu.sync_copy(rows, tbl.at[dbuf], add=True)
            return carry

        lax.fori_loop(0, NBLK, body, 0)
        plsc.subcore_barrier()
        pltpu.sync_copy(tbl.at[pl.ds(rbase, STRIPE), :],
                        out_hbm.at[k].at[pl.ds(rbase, STRIPE), :])
        plsc.subcore_barrier()


def _mm1_body(x_ref, w_ref, dinv_ref, o_ref):
    h = jnp.dot(x_ref[...], w_ref[...], preferred_element_type=jnp.float32)
    o_ref[...] = h * dinv_ref[...][:, None]


def _p3_body(a_ref, g_ref, dinv_ref, w2_ref, b1_ref, o_ref):
    dv = dinv_ref[...][:, None]
    r1 = jnp.maximum(dv * (a_ref[...] + g_ref[...]) + b1_ref[...][None, :],
                     0.0)
    h2 = jnp.dot(r1, w2_ref[...], preferred_element_type=jnp.float32)
    o_ref[...] = h2 * dv


def _p5_body(a_ref, g_ref, dinv_ref, s_ref, w3_ref, b2_ref, b3_ref,
             o_ref, acc):
    i = pl.program_id(0)
    dvec = dinv_ref[...]
    dv = dvec[:, None]
    r2 = jnp.maximum(dv * (a_ref[...] + g_ref[...]) + b2_ref[...][None, :],
                     0.0)
    wv = (dvec * (dvec + s_ref[...]))[None, :]
    part = jnp.dot(wv, r2, preferred_element_type=jnp.float32)

    @pl.when(i == 0)
    def _():
        acc[...] = part

    @pl.when(i > 0)
    def _():
        acc[...] += part

    @pl.when(i == pl.num_programs(0) - 1)
    def _():
        o_ref[...] = (jnp.dot(acc[...], w3_ref[...],
                              preferred_element_type=jnp.float32) / N
                      + b3_ref[...][None, :])


def _tc_mm1(x, w1p, dinv):
    return pl.pallas_call(
        _mm1_body,
        grid=(GRID,),
        in_specs=[
            pl.BlockSpec((BLK, 50), lambda i: (i, 0)),
            pl.BlockSpec((50, D), lambda i: (0, 0)),
            pl.BlockSpec((BLK,), lambda i: (i,)),
        ],
        out_specs=pl.BlockSpec((BLK, D), lambda i: (i, 0)),
        out_shape=jax.ShapeDtypeStruct((NT, D), jnp.float32),
    )(x, w1p, dinv)


def _tc_p3(a1, g1, dinv, w2p, b1p):
    return pl.pallas_call(
        _p3_body,
        grid=(GRID,),
        in_specs=[
            pl.BlockSpec((BLK, D), lambda i: (i, 0)),
            pl.BlockSpec((BLK, D), lambda i: (i, 0)),
            pl.BlockSpec((BLK,), lambda i: (i,)),
            pl.BlockSpec((D, D), lambda i: (0, 0)),
            pl.BlockSpec((D,), lambda i: (0,)),
        ],
        out_specs=pl.BlockSpec((BLK, D), lambda i: (i, 0)),
        out_shape=jax.ShapeDtypeStruct((NT, D), jnp.float32),
    )(a1, g1, dinv, w2p, b1p)


def _tc_p5(a2, g2, dinv, s, w3p, b2p, b3p):
    return pl.pallas_call(
        _p5_body,
        grid=(GRID,),
        in_specs=[
            pl.BlockSpec((BLK, D), lambda i: (i, 0)),
            pl.BlockSpec((BLK, D), lambda i: (i, 0)),
            pl.BlockSpec((BLK,), lambda i: (i,)),
            pl.BlockSpec((BLK,), lambda i: (i,)),
            pl.BlockSpec((D, 128), lambda i: (0, 0)),
            pl.BlockSpec((D,), lambda i: (0,)),
            pl.BlockSpec((128,), lambda i: (0,)),
        ],
        out_specs=pl.BlockSpec((1, 128), lambda i: (0, 0)),
        out_shape=jax.ShapeDtypeStruct((1, 128), jnp.float32),
        scratch_shapes=[pltpu.VMEM((1, D), jnp.float32)],
    )(a2, g2, dinv, s, w3p, b2p, b3p)


def _to_chunks(g64):
    gc = jnp.transpose(g64[:N].reshape(N, C, CW), (1, 0, 2))
    return jnp.pad(gc, ((0, 0), (0, NP - N), (0, 0)))


def _from_chunks(ac):
    a = jnp.transpose(ac[:, :N, :], (1, 0, 2)).reshape(N, D)
    return jnp.pad(a, ((0, NT - N), (0, 0)))


def kernel(x, edge_index, edge_attr, W1, b1, W2, b2, W3, b3):
    src = edge_index[0].astype(jnp.int32)
    dst = edge_index[1].astype(jnp.int32)
    pad = jnp.full((EP - E,), N, jnp.int32)
    srcp = jnp.concatenate([src, pad])
    dstp = jnp.concatenate([dst, pad])

    zeros1 = jnp.zeros((NP,), jnp.float32)
    zeros2 = jnp.zeros((NP, CW), jnp.float32)

    w1p = jnp.pad(W1, ((0, 0), (0, D - 50)))
    w2p = jnp.pad(W2, ((0, D - 50), (0, D - 50)))
    w3p = jnp.pad(W3, ((0, D - 50), (0, 128 - 100)))
    b1p = jnp.pad(b1, (0, D - 50))
    b2p = jnp.pad(b2, (0, D - 50))
    b3p = jnp.pad(b3, (0, 128 - 100))

    degp = _sc_deg(dstp, zeros1)
    dinv_full = lax.rsqrt(degp[0] + degp[1] + 1.0)
    dinv = jnp.pad(dinv_full[:N], (0, NT - N))

    sp = _sc_s(srcp, dstp, dinv_full, zeros1)
    s = jnp.pad((sp[0] + sp[1])[:N], (0, NT - N))

    xt = jnp.pad(x, ((0, NT - N), (0, 0)))
    g1 = _tc_mm1(xt, w1p, dinv)
    a1 = _from_chunks(_sc_agg(_to_chunks(g1), srcp, dstp, zeros2))
    g2 = _tc_p3(a1, g1, dinv, w2p, b1p)
    a2 = _from_chunks(_sc_agg(_to_chunks(g2), srcp, dstp, zeros2))
    out = _tc_p5(a2, g2, dinv, s, w3p, b2p, b3p)
    return out[:, :100]


# speedup vs baseline: 9.7154x; 1.4794x over previous
import functools

import jax
import jax.numpy as jnp
from jax import lax
from jax.experimental import pallas as pl
from jax.experimental.pallas import tpu as pltpu
from jax.experimental.pallas import tpu_sc as plsc

N = 100000
E = 1600000
NP = 100352
STRIPE = NP // 16
EP = 1671168
B = 128
GB = 8
NBLK = EP // (16 * B)
NBLK2 = EP // (32 * B)
NG = NBLK // GB
NG2 = NBLK2 // GB
EB = EP // B
CW = 16
C = 4
D = 64
NT = 102400
BLK = 1024
GRID = NT // BLK

_mesh = plsc.VectorSubcoreMesh(core_axis_name="c", subcore_axis_name="s")


@functools.partial(
    pl.kernel,
    out_type=jax.ShapeDtypeStruct((2, NP), jnp.float32),
    mesh=_mesh,
    compiler_params=pltpu.CompilerParams(use_tc_tiling_on_sc=False),
    scratch_types=[
        pltpu.VMEM_SHARED((NP,), jnp.float32),
        pltpu.VMEM((GB, B), jnp.int32),
        pltpu.VMEM((B,), jnp.float32),
        pltpu.SemaphoreType.DMA,
    ],
)
def _sc_deg(dst_hbm, zeros1_hbm, out_hbm, tbl, dbuf, ones, sem):
    cc = lax.axis_index("c")
    ss = lax.axis_index("s")
    rbase = ss * STRIPE
    for i in range(B // 16):
        ones[pl.ds(i * 16, 16)] = jnp.full((16,), 1.0, jnp.float32)
    pltpu.sync_copy(zeros1_hbm.at[pl.ds(rbase, STRIPE)],
                    tbl.at[pl.ds(rbase, STRIPE)])
    plsc.subcore_barrier()
    gbase = (cc * 16 + ss) * NBLK2

    def body(g, carry):
        grow = gbase + g * GB
        pltpu.sync_copy(dst_hbm.at[pl.ds(grow, GB), :], dbuf)
        descs = [pltpu.async_copy(ones, tbl.at[dbuf.at[j]], sem, add=True)
                 for j in range(GB)]
        for d in descs:
            d.wait()
        return carry

    lax.fori_loop(0, NG2, body, 0)
    plsc.subcore_barrier()
    pltpu.sync_copy(tbl.at[pl.ds(rbase, STRIPE)],
                    out_hbm.at[cc].at[pl.ds(rbase, STRIPE)])


@functools.partial(
    pl.kernel,
    out_type=jax.ShapeDtypeStruct((2, NP), jnp.float32),
    mesh=_mesh,
    compiler_params=pltpu.CompilerParams(use_tc_tiling_on_sc=False),
    scratch_types=[
        pltpu.VMEM_SHARED((NP,), jnp.float32),
        pltpu.VMEM((GB, B), jnp.int32),
        pltpu.VMEM((GB, B), jnp.int32),
        pltpu.VMEM((GB, B), jnp.float32),
        pltpu.SemaphoreType.DMA,
        pltpu.SemaphoreType.DMA,
    ],
)
def _sc_s(src_hbm, dst_hbm, dinv_hbm, zeros1_hbm, out_hbm,
          tbl, sbuf, dbuf, dvals, semg, sems):
    cc = lax.axis_index("c")
    ss = lax.axis_index("s")
    rbase = ss * STRIPE
    pltpu.sync_copy(zeros1_hbm.at[pl.ds(rbase, STRIPE)],
                    tbl.at[pl.ds(rbase, STRIPE)])
    plsc.subcore_barrier()
    gbase = (cc * 16 + ss) * NBLK2

    def body(g, carry):
        grow = gbase + g * GB
        pltpu.sync_copy(src_hbm.at[pl.ds(grow, GB), :], sbuf)
        pltpu.sync_copy(dst_hbm.at[pl.ds(grow, GB), :], dbuf)
        gd = [pltpu.async_copy(dinv_hbm.at[dbuf.at[j]], dvals.at[j], semg)
              for j in range(GB)]
        for d in gd:
            d.wait()
        sd = [pltpu.async_copy(dvals.at[j], tbl.at[sbuf.at[j]], sems,
                               add=True) for j in range(GB)]
        for d in sd:
            d.wait()
        return carry

    lax.fori_loop(0, NG2, body, 0)
    plsc.subcore_barrier()
    pltpu.sync_copy(tbl.at[pl.ds(rbase, STRIPE)],
                    out_hbm.at[cc].at[pl.ds(rbase, STRIPE)])


@functools.partial(
    pl.kernel,
    out_type=jax.ShapeDtypeStruct((C, NP, CW), jnp.float32),
    mesh=_mesh,
    compiler_params=pltpu.CompilerParams(use_tc_tiling_on_sc=False),
    scratch_types=[
        pltpu.VMEM_SHARED((NP, CW), jnp.float32),
        pltpu.VMEM((GB, B), jnp.int32),
        pltpu.VMEM((GB, B), jnp.int32),
        pltpu.VMEM((GB, B, CW), jnp.float32),
        pltpu.SemaphoreType.DMA,
        pltpu.SemaphoreType.DMA,
    ],
)
def _sc_agg(g_hbm, src_hbm, dst_hbm, zeros2_hbm, out_hbm,
            tbl, sbuf, dbuf, rows, semg, sems):
    cc = lax.axis_index("c")
    ss = lax.axis_index("s")
    rbase = ss * STRIPE
    for p in range(2):
        k = cc * 2 + p
        pltpu.sync_copy(zeros2_hbm.at[pl.ds(rbase, STRIPE), :],
                        tbl.at[pl.ds(rbase, STRIPE), :])
        plsc.subcore_barrier()
        gbase = ss * NBLK

        def body(g, carry):
            grow = gbase + g * GB
            pltpu.sync_copy(src_hbm.at[pl.ds(grow, GB), :], sbuf)
            pltpu.sync_copy(dst_hbm.at[pl.ds(grow, GB), :], dbuf)
            gd = [pltpu.async_copy(g_hbm.at[k].at[sbuf.at[j]], rows.at[j],
                                   semg) for j in range(GB)]
            for d in gd:
                d.wait()
            sd = [pltpu.async_copy(rows.at[j], tbl.at[dbuf.at[j]], sems,
                                   add=True) for j in range(GB)]
            for d in sd:
                d.wait()
            return carry

        lax.fori_loop(0, NG, body, 0)
        plsc.subcore_barrier()
        pltpu.sync_copy(tbl.at[pl.ds(rbase, STRIPE), :],
                        out_hbm.at[k].at[pl.ds(rbase, STRIPE), :])
        plsc.subcore_barrier()


def _mm1_body(x_ref, w_ref, dinv_ref, o_ref):
    h = jnp.dot(x_ref[...], w_ref[...], preferred_element_type=jnp.float32)
    o_ref[...] = h * dinv_ref[...][:, None]


def _p3_body(a_ref, g_ref, dinv_ref, w2_ref, b1_ref, o_ref):
    dv = dinv_ref[...][:, None]
    r1 = jnp.maximum(dv * (a_ref[...] + g_ref[...]) + b1_ref[...][None, :],
                     0.0)
    h2 = jnp.dot(r1, w2_ref[...], preferred_element_type=jnp.float32)
    o_ref[...] = h2 * dv


def _p5_body(a_ref, g_ref, dinv_ref, s_ref, w3_ref, b2_ref, b3_ref,
             o_ref, acc):
    i = pl.program_id(0)
    dvec = dinv_ref[...]
    dv = dvec[:, None]
    r2 = jnp.maximum(dv * (a_ref[...] + g_ref[...]) + b2_ref[...][None, :],
                     0.0)
    wv = (dvec * (dvec + s_ref[...]))[None, :]
    part = jnp.dot(wv, r2, preferred_element_type=jnp.float32)

    @pl.when(i == 0)
    def _():
        acc[...] = part

    @pl.when(i > 0)
    def _():
        acc[...] += part

    @pl.when(i == pl.num_programs(0) - 1)
    def _():
        o_ref[...] = (jnp.dot(acc[...], w3_ref[...],
                              preferred_element_type=jnp.float32) / N
                      + b3_ref[...][None, :])


def _tc_mm1(x, w1p, dinv):
    return pl.pallas_call(
        _mm1_body,
        grid=(GRID,),
        in_specs=[
            pl.BlockSpec((BLK, 50), lambda i: (i, 0)),
            pl.BlockSpec((50, D), lambda i: (0, 0)),
            pl.BlockSpec((BLK,), lambda i: (i,)),
        ],
        out_specs=pl.BlockSpec((BLK, D), lambda i: (i, 0)),
        out_shape=jax.ShapeDtypeStruct((NT, D), jnp.float32),
    )(x, w1p, dinv)


def _tc_p3(a1, g1, dinv, w2p, b1p):
    return pl.pallas_call(
        _p3_body,
        grid=(GRID,),
        in_specs=[
            pl.BlockSpec((BLK, D), lambda i: (i, 0)),
            pl.BlockSpec((BLK, D), lambda i: (i, 0)),
            pl.BlockSpec((BLK,), lambda i: (i,)),
            pl.BlockSpec((D, D), lambda i: (0, 0)),
            pl.BlockSpec((D,), lambda i: (0,)),
        ],
        out_specs=pl.BlockSpec((BLK, D), lambda i: (i, 0)),
        out_shape=jax.ShapeDtypeStruct((NT, D), jnp.float32),
    )(a1, g1, dinv, w2p, b1p)


def _tc_p5(a2, g2, dinv, s, w3p, b2p, b3p):
    return pl.pallas_call(
        _p5_body,
        grid=(GRID,),
        in_specs=[
            pl.BlockSpec((BLK, D), lambda i: (i, 0)),
            pl.BlockSpec((BLK, D), lambda i: (i, 0)),
            pl.BlockSpec((BLK,), lambda i: (i,)),
            pl.BlockSpec((BLK,), lambda i: (i,)),
            pl.BlockSpec((D, 128), lambda i: (0, 0)),
            pl.BlockSpec((D,), lambda i: (0,)),
            pl.BlockSpec((128,), lambda i: (0,)),
        ],
        out_specs=pl.BlockSpec((1, 128), lambda i: (0, 0)),
        out_shape=jax.ShapeDtypeStruct((1, 128), jnp.float32),
        scratch_shapes=[pltpu.VMEM((1, D), jnp.float32)],
    )(a2, g2, dinv, s, w3p, b2p, b3p)


def _to_chunks(g64):
    gc = jnp.transpose(g64[:N].reshape(N, C, CW), (1, 0, 2))
    return jnp.pad(gc, ((0, 0), (0, NP - N), (0, 0)))


def _from_chunks(ac):
    a = jnp.transpose(ac[:, :N, :], (1, 0, 2)).reshape(N, D)
    return jnp.pad(a, ((0, NT - N), (0, 0)))


def kernel(x, edge_index, edge_attr, W1, b1, W2, b2, W3, b3):
    src = edge_index[0].astype(jnp.int32)
    dst = edge_index[1].astype(jnp.int32)
    pad = jnp.full((EP - E,), N, jnp.int32)
    srcp = jnp.concatenate([src, pad]).reshape(EB, B)
    dstp = jnp.concatenate([dst, pad]).reshape(EB, B)

    zeros1 = jnp.zeros((NP,), jnp.float32)
    zeros2 = jnp.zeros((NP, CW), jnp.float32)

    w1p = jnp.pad(W1, ((0, 0), (0, D - 50)))
    w2p = jnp.pad(W2, ((0, D - 50), (0, D - 50)))
    w3p = jnp.pad(W3, ((0, D - 50), (0, 128 - 100)))
    b1p = jnp.pad(b1, (0, D - 50))
    b2p = jnp.pad(b2, (0, D - 50))
    b3p = jnp.pad(b3, (0, 128 - 100))

    degp = _sc_deg(dstp, zeros1)
    dinv_full = lax.rsqrt(degp[0] + degp[1] + 1.0)
    dinv = jnp.pad(dinv_full[:N], (0, NT - N))

    sp = _sc_s(srcp, dstp, dinv_full, zeros1)
    s = jnp.pad((sp[0] + sp[1])[:N], (0, NT - N))

    xt = jnp.pad(x, ((0, NT - N), (0, 0)))
    g1 = _tc_mm1(xt, w1p, dinv)
    a1 = _from_chunks(_sc_agg(_to_chunks(g1), srcp, dstp, zeros2))
    g2 = _tc_p3(a1, g1, dinv, w2p, b1p)
    a2 = _from_chunks(_sc_agg(_to_chunks(g2), srcp, dstp, zeros2))
    out = _tc_p5(a2, g2, dinv, s, w3p, b2p, b3p)
    return out[:, :100]
